# baseline (speedup 1.0000x reference)
import functools

import jax
import jax.numpy as jnp
from jax import lax
from jax.experimental import pallas as pl
from jax.experimental.pallas import tpu as pltpu
from jax.experimental.pallas import tpu_sc as plsc

C = 26
V = 100000
D = 32
B = 16384

NC = 2
NS = 16
NW = NC * NS

SUB = 512
NSUB = 196
VPAD = NSUB * SUB
SPG = 28
NVG = NSUB // SPG

SPLITS = (7, 7, 6, 6)
BC = 512
NBC = B // BC
NTL = BC // 128
SROW = 137


def _k1_body(in_ref, o_ref):
    blk = in_ref[0]
    for s in range(SPG):
        in4 = jnp.concatenate(
            [blk[:, s * 512 + q * 128: s * 512 + (q + 1) * 128]
             for q in range(4)], axis=0)
        o_ref[s * 128:(s + 1) * 128, :] = in4.T


def _k2_body(c0, ipw, idx_hbm, w1_hbm, out_hbm, idx_v, rows_v, stag_v, t_v,
             sems):
    wid = lax.axis_index("s") * NC + lax.axis_index("c")
    iota16 = lax.iota(jnp.int32, 16)
    iota_s = iota16 * SROW

    pltpu.sync_copy(idx_hbm.at[pl.ds(c0 * B + wid * ipw * BC, ipw * BC)],
                    idx_v)

    @plsc.parallel_loop(0, ipw * BC, step=16, unroll=4)
    def _(i):
        item = wid * ipw + i // BC
        cbase = (item // NBC) * VPAD
        v = idx_v.at[pl.ds(i, 16)][...]
        j = (((v >> 9) << 9) + ((v & 127) << 2) + ((v >> 7) & 3)) + cbase
        idx_v.at[pl.ds(i, 16)][...] = j

    def launch_gather(t, p):
        return pltpu.async_copy(
            w1_hbm.at[idx_v.at[pl.ds(t * BC, BC)]], rows_v.at[p],
            sems.at[p])

    def transform(t, p):
        item = wid * ipw + t
        c = item // NBC
        b0 = (item % NBC) * BC
        for h in range(2):
            @plsc.parallel_loop(0, NTL)
            def _(btl):
                @plsc.parallel_loop(0, 128, unroll=4)
                def _(bl):
                    val = rows_v.at[
                        p, btl * 128 + bl, pl.ds(h * 16, 16)][...]
                    plsc.store_scatter(
                        stag_v.at[btl], [iota_s + bl], val)

                for hh in range(2):
                    @plsc.parallel_loop(0, 16, unroll=4)
                    def _(it):
                        di = it // 2
                        bi0 = (it % 2) * 64
                        for u in range(0, 64, 16):
                            bi = bi0 + u
                            t_v.at[hh, btl, di, pl.ds(bi, 16)][...] = \
                                stag_v.at[btl, pl.ds(
                                    (hh * 8 + di) * SROW + bi, 16)][...]

            for hh in range(2):
                pltpu.sync_copy(
                    t_v.at[hh],
                    out_hbm.at[c, 2 * h + hh, pl.ds(b0 // 128, NTL)])

    copies = {}
    copies[0] = launch_gather(0, 0)
    for t in range(ipw):
        p = t % 2
        if t + 1 < ipw:
            copies[t + 1] = launch_gather(t + 1, 1 - p)
        copies[t].wait()
        transform(t, p)


def _make_stage(c0, ch):
    ipw = ch * NBC // NW
    assert ipw * NW == ch * NBC
    mesh = plsc.VectorSubcoreMesh(core_axis_name="c", subcore_axis_name="s")
    k2 = functools.partial(
        pl.kernel,
        out_type=jax.ShapeDtypeStruct((ch, 4, 128, 8, 128), jnp.float32),
        mesh=mesh,
        scratch_types=[
            pltpu.VMEM((ipw * BC,), jnp.int32),
            pltpu.VMEM((2, BC, D), jnp.float32),
            pltpu.VMEM((NTL, 16 * SROW), jnp.float32),
            pltpu.VMEM((2, NTL, 8, 128), jnp.float32),
            pltpu.SemaphoreType.DMA((2,)),
        ],
        compiler_params=pltpu.CompilerParams(
            use_tc_tiling_on_sc=False, needs_layout_passes=False),
    )(functools.partial(_k2_body, c0, ipw))

    def stage(Wt, idxT):
        w1 = pl.pallas_call(
            _k1_body,
            grid=(ch, NVG),
            in_specs=[pl.BlockSpec((1, D, SPG * SUB),
                                   lambda c, v: (c0 + c, 0, v))],
            out_specs=pl.BlockSpec((SPG * 128, 128),
                                   lambda c, v: (c * NVG + v, 0)),
            out_shape=jax.ShapeDtypeStruct((ch * NSUB * 128, 128),
                                           jnp.float32),
            compiler_params=pltpu.CompilerParams(
                dimension_semantics=("parallel", "parallel")),
        )(Wt)
        w1r = w1.reshape(ch * VPAD, D)
        return k2(idxT.reshape(C * B), w1r)

    return stage


_stages = []
_c0 = 0
for _ch in SPLITS:
    _stages.append(_make_stage(_c0, _ch))
    _c0 += _ch


def kernel(input, W):
    Wt = jnp.transpose(W, (0, 2, 1))
    idxT = jnp.transpose(input)
    F = jnp.concatenate([s(Wt, idxT) for s in _stages], axis=0)
    return F.transpose(2, 4, 0, 1, 3).reshape(B, C, D)


# speedup vs baseline: 1.0019x; 1.0019x over previous
import functools

import jax
import jax.numpy as jnp
from jax import lax
from jax.experimental import pallas as pl
from jax.experimental.pallas import tpu as pltpu
from jax.experimental.pallas import tpu_sc as plsc

C = 26
V = 100000
D = 32
B = 16384

NC = 2
NS = 16
NW = NC * NS

SUB = 512
NSUB = 196
VPAD = NSUB * SUB
SPG = 28
NVG = NSUB // SPG

SPLITS = (7, 7, 6, 6)
BC = 512
NBC = B // BC
NTL = BC // 128
SROW = 137


def _k1_body(in_ref, o_ref):
    blk = in_ref[0]
    for s in range(SPG):
        in4 = jnp.concatenate(
            [blk[:, s * 512 + q * 128: s * 512 + (q + 1) * 128]
             for q in range(4)], axis=0)
        o_ref[s * 128:(s + 1) * 128, :] = in4.T


def _k2_body(c0, ipw, idx_hbm, w1_hbm, out_hbm, idx_v, rows_v, stag_v, t_v,
             sems):
    wid = lax.axis_index("s") * NC + lax.axis_index("c")
    iota16 = lax.iota(jnp.int32, 16)
    iota_s = iota16 * SROW

    pltpu.sync_copy(idx_hbm.at[pl.ds(c0 * B + wid * ipw * BC, ipw * BC)],
                    idx_v)

    @plsc.parallel_loop(0, ipw * BC, step=16, unroll=4)
    def _(i):
        item = wid * ipw + i // BC
        cbase = (item // NBC) * VPAD
        v = idx_v.at[pl.ds(i, 16)][...]
        j = (((v >> 9) << 9) + ((v & 127) << 2) + ((v >> 7) & 3)) + cbase
        idx_v.at[pl.ds(i, 16)][...] = j

    def launch_gather(t, p):
        return pltpu.async_copy(
            w1_hbm.at[idx_v.at[pl.ds(t * BC, BC)]], rows_v.at[p],
            sems.at[p])

    def transform(t, p):
        item = wid * ipw + t
        c = item // NBC
        b0 = (item % NBC) * BC
        for h in range(2):
            @pl.loop(0, NTL)
            def _(btl):
                @plsc.parallel_loop(0, 128, unroll=4)
                def _(bl):
                    val = rows_v.at[
                        p, btl * 128 + bl, pl.ds(h * 16, 16)][...]
                    plsc.store_scatter(stag_v, [iota_s + bl], val)

                for hh in range(2):
                    @plsc.parallel_loop(0, 16, unroll=4)
                    def _(it):
                        di = it // 2
                        bi0 = (it % 2) * 64
                        for u in range(0, 64, 16):
                            bi = bi0 + u
                            t_v.at[hh, btl, di, pl.ds(bi, 16)][...] = \
                                stag_v.at[pl.ds(
                                    (hh * 8 + di) * SROW + bi, 16)][...]

            for hh in range(2):
                pltpu.sync_copy(
                    t_v.at[hh],
                    out_hbm.at[c, 2 * h + hh, pl.ds(b0 // 128, NTL)])

    copies = {}
    copies[0] = launch_gather(0, 0)
    for t in range(ipw):
        p = t % 2
        if t + 1 < ipw:
            copies[t + 1] = launch_gather(t + 1, 1 - p)
        copies[t].wait()
        transform(t, p)


def _make_stage(c0, ch):
    ipw = ch * NBC // NW
    assert ipw * NW == ch * NBC
    mesh = plsc.VectorSubcoreMesh(core_axis_name="c", subcore_axis_name="s")
    k2 = functools.partial(
        pl.kernel,
        out_type=jax.ShapeDtypeStruct((ch, 4, 128, 8, 128), jnp.float32),
        mesh=mesh,
        scratch_types=[
            pltpu.VMEM((ipw * BC,), jnp.int32),
            pltpu.VMEM((2, BC, D), jnp.float32),
            pltpu.VMEM((16 * SROW,), jnp.float32),
            pltpu.VMEM((2, NTL, 8, 128), jnp.float32),
            pltpu.SemaphoreType.DMA((2,)),
        ],
        compiler_params=pltpu.CompilerParams(
            use_tc_tiling_on_sc=False, needs_layout_passes=False),
    )(functools.partial(_k2_body, c0, ipw))

    def stage(Wt, idxT):
        w1 = pl.pallas_call(
            _k1_body,
            grid=(ch, NVG),
            in_specs=[pl.BlockSpec((1, D, SPG * SUB),
                                   lambda c, v: (c0 + c, 0, v))],
            out_specs=pl.BlockSpec((SPG * 128, 128),
                                   lambda c, v: (c * NVG + v, 0)),
            out_shape=jax.ShapeDtypeStruct((ch * NSUB * 128, 128),
                                           jnp.float32),
            compiler_params=pltpu.CompilerParams(
                dimension_semantics=("parallel", "parallel")),
        )(Wt)
        w1r = w1.reshape(ch * VPAD, D)
        return k2(idxT.reshape(C * B), w1r)

    return stage


_stages = []
_c0 = 0
for _ch in SPLITS:
    _stages.append(_make_stage(_c0, _ch))
    _c0 += _ch


def kernel(input, W):
    Wt = jnp.transpose(W, (0, 2, 1))
    idxT = jnp.transpose(input)
    F = jnp.concatenate([s(Wt, idxT) for s in _stages], axis=0)
    return F.transpose(2, 4, 0, 1, 3).reshape(B, C, D)
